# baseline (speedup 1.0000x reference)
import functools

import jax
import jax.numpy as jnp
from jax import lax
from jax.experimental import pallas as pl
from jax.experimental.pallas import tpu as pltpu
from jax.experimental.pallas import tpu_sc as plsc

_LANES = 16
_NBUF = 2
_R = 4


def _make_sc_add(B, S, D, rows_per_w):
    R = _R
    n_chunks = rows_per_w // R
    n_groups = n_chunks // _NBUF
    mesh = plsc.VectorSubcoreMesh(core_axis_name="c", subcore_axis_name="s")
    NC = mesh.num_cores

    @functools.partial(
        pl.kernel,
        out_type=jax.ShapeDtypeStruct((B, S, D), jnp.float32),
        mesh=mesh,
        scratch_types=[
            pltpu.VMEM((_NBUF, R, D), jnp.float32),
            pltpu.VMEM((_NBUF, B, R, D), jnp.float32),
            pltpu.VMEM((_NBUF, B, R, D), jnp.float32),
            pltpu.SemaphoreType.DMA,
            pltpu.SemaphoreType.DMA,
            pltpu.SemaphoreType.DMA,
            pltpu.SemaphoreType.DMA,
        ],
    )
    def sc_add(x_hbm, pos_hbm, out_hbm, pbuf, xbuf, obuf, in0, in1, out0, out1):
        in_sems = (in0, in1)
        out_sems = (out0, out1)
        wid = lax.axis_index("s") * NC + lax.axis_index("c")
        base = wid * rows_per_w

        def in_descs(ci, k):
            s0 = base + ci * R
            descs = [
                pltpu.make_async_copy(
                    pos_hbm.at[pl.ds(s0, R)], pbuf.at[k], in_sems[k]
                )
            ]
            for b in range(B):
                descs.append(
                    pltpu.make_async_copy(
                        x_hbm.at[b, pl.ds(s0, R)], xbuf.at[k, b], in_sems[k]
                    )
                )
            return descs

        def out_descs(ci, k):
            s0 = base + ci * R
            return [
                pltpu.make_async_copy(
                    obuf.at[k, b], out_hbm.at[b, pl.ds(s0, R)], out_sems[k]
                )
                for b in range(B)
            ]

        def start_in(ci, k):
            for d in in_descs(ci, k):
                d.start()

        def wait_in(ci, k):
            for d in in_descs(ci, k):
                d.wait()

        def start_out(ci, k):
            for d in out_descs(ci, k):
                d.start()

        def wait_out(ci, k):
            for d in out_descs(ci, k):
                d.wait()

        def compute(k):
            unroll = 4
            for r in range(R):

                def col(i, carry3):
                    for u in range(unroll):
                        c = (i * unroll + u) * _LANES
                        p = pbuf[k, r, pl.ds(c, _LANES)]
                        for b in range(B):
                            obuf[k, b, r, pl.ds(c, _LANES)] = (
                                xbuf[k, b, r, pl.ds(c, _LANES)] + p
                            )
                    return carry3

                lax.fori_loop(0, D // (_LANES * unroll), col, 0)

        for k in range(_NBUF):
            start_in(k, k)

        def group(g, carry):
            for k in range(_NBUF):
                ci = g * _NBUF + k
                wait_in(ci, k)

                @pl.when(g > 0)
                def _():
                    wait_out(ci - _NBUF, k)

                compute(k)
                start_out(ci, k)

                @pl.when(g < n_groups - 1)
                def _():
                    start_in(ci + _NBUF, k)

            return carry

        lax.fori_loop(0, n_groups, group, 0)
        for k in range(_NBUF):
            wait_out(n_chunks - _NBUF + k, k)

    return sc_add


def kernel(x, pos_embedding):
    B, S, D = x.shape
    NW = 32
    rows_per_w = S // NW
    sc_add = _make_sc_add(B, S, D, rows_per_w)
    return sc_add(x, pos_embedding[:S])


# speedup vs baseline: 1.5187x; 1.5187x over previous
import functools

import jax
import jax.numpy as jnp
from jax import lax
from jax.experimental import pallas as pl
from jax.experimental.pallas import tpu as pltpu
from jax.experimental.pallas import tpu_sc as plsc

_LANES = 16
_NBUF = 2
_R = 8


def _make_sc_add(B, S, D, rows_per_w):
    R = _R
    n_chunks = rows_per_w // R
    n_groups = n_chunks // _NBUF
    mesh = plsc.VectorSubcoreMesh(core_axis_name="c", subcore_axis_name="s")
    NC = mesh.num_cores

    @functools.partial(
        pl.kernel,
        out_type=jax.ShapeDtypeStruct((B, S, D), jnp.float32),
        mesh=mesh,
        scratch_types=[
            pltpu.VMEM((_NBUF, R, D), jnp.float32),
            pltpu.VMEM((_NBUF, B, R, D), jnp.float32),
            pltpu.SemaphoreType.DMA,
            pltpu.SemaphoreType.DMA,
            pltpu.SemaphoreType.DMA,
            pltpu.SemaphoreType.DMA,
        ],
    )
    def sc_add(x_hbm, pos_hbm, out_hbm, pbuf, obuf, in0, in1, out0, out1):
        in_sems = (in0, in1)
        out_sems = (out0, out1)
        wid = lax.axis_index("s") * NC + lax.axis_index("c")
        base = wid * rows_per_w

        def in_descs(ci, k):
            s0 = base + ci * R
            descs = [
                pltpu.make_async_copy(
                    pos_hbm.at[pl.ds(s0, R)], pbuf.at[k], in_sems[k]
                )
            ]
            for b in range(B):
                descs.append(
                    pltpu.make_async_copy(
                        x_hbm.at[b, pl.ds(s0, R)], obuf.at[k, b], in_sems[k]
                    )
                )
            return descs

        def out_descs(ci, k):
            s0 = base + ci * R
            return [
                pltpu.make_async_copy(
                    obuf.at[k, b], out_hbm.at[b, pl.ds(s0, R)], out_sems[k]
                )
                for b in range(B)
            ]

        def start_in(ci, k):
            for d in in_descs(ci, k):
                d.start()

        def wait_in(ci, k):
            for d in in_descs(ci, k):
                d.wait()

        def start_out(ci, k):
            for d in out_descs(ci, k):
                d.start()

        def wait_out(ci, k):
            for d in out_descs(ci, k):
                d.wait()

        def compute(k):
            for r in range(R):

                def col(i, carry3):
                    c = i * _LANES
                    p = pbuf[k, r, pl.ds(c, _LANES)]
                    for b in range(B):
                        plsc.addupdate(obuf.at[k, b, r, pl.ds(c, _LANES)], p)
                    return carry3

                lax.fori_loop(0, D // _LANES, col, 0)

        for k in range(_NBUF):
            start_in(k, k)

        def group(g, carry):
            for k in range(_NBUF):
                ci = g * _NBUF + k
                wait_in(ci, k)
                compute(k)
                start_out(ci, k)

                @pl.when(g < n_groups - 1)
                def _():
                    wait_out(ci, k)
                    start_in(ci + _NBUF, k)

            return carry

        lax.fori_loop(0, n_groups, group, 0)
        for k in range(_NBUF):
            wait_out(n_chunks - _NBUF + k, k)

    return sc_add


def kernel(x, pos_embedding):
    B, S, D = x.shape
    NW = 32
    rows_per_w = S // NW
    sc_add = _make_sc_add(B, S, D, rows_per_w)
    return sc_add(x, pos_embedding[:S])


# speedup vs baseline: 1.7894x; 1.1782x over previous
import functools

import jax
import jax.numpy as jnp
from jax import lax
from jax.experimental import pallas as pl
from jax.experimental.pallas import tpu as pltpu
from jax.experimental.pallas import tpu_sc as plsc

_LANES = 16
_NBUF = 3
_R = 4


def _make_sc_add(B, S, D, rows_per_w):
    R = _R
    n_chunks = rows_per_w // R
    n_groups = n_chunks // _NBUF
    mesh = plsc.VectorSubcoreMesh(core_axis_name="c", subcore_axis_name="s")
    NC = mesh.num_cores

    @functools.partial(
        pl.kernel,
        out_type=jax.ShapeDtypeStruct((B, S, D), jnp.float32),
        mesh=mesh,
        scratch_types=[
            pltpu.VMEM((_NBUF, R, D), jnp.float32),
            pltpu.VMEM((_NBUF, B, R, D), jnp.float32),
            pltpu.VMEM((_NBUF, B, R, D), jnp.float32),
        ]
        + [pltpu.SemaphoreType.DMA] * (2 * _NBUF),
    )
    def sc_add(x_hbm, pos_hbm, out_hbm, pbuf, xbuf, obuf, *sems):
        in_sems = sems[:_NBUF]
        out_sems = sems[_NBUF:]
        wid = lax.axis_index("s") * NC + lax.axis_index("c")
        base = wid * rows_per_w

        def in_descs(ci, k):
            s0 = base + ci * R
            descs = [
                pltpu.make_async_copy(
                    pos_hbm.at[pl.ds(s0, R)], pbuf.at[k], in_sems[k]
                )
            ]
            for b in range(B):
                descs.append(
                    pltpu.make_async_copy(
                        x_hbm.at[b, pl.ds(s0, R)], xbuf.at[k, b], in_sems[k]
                    )
                )
            return descs

        def out_descs(ci, k):
            s0 = base + ci * R
            return [
                pltpu.make_async_copy(
                    obuf.at[k, b], out_hbm.at[b, pl.ds(s0, R)], out_sems[k]
                )
                for b in range(B)
            ]

        def start_in(ci, k):
            for d in in_descs(ci, k):
                d.start()

        def wait_in(ci, k):
            for d in in_descs(ci, k):
                d.wait()

        def start_out(ci, k):
            for d in out_descs(ci, k):
                d.start()

        def wait_out(ci, k):
            for d in out_descs(ci, k):
                d.wait()

        def compute(k):
            for r in range(R):

                def col(i, carry3):
                    c = i * _LANES
                    p = pbuf[k, r, pl.ds(c, _LANES)]
                    for b in range(B):
                        obuf[k, b, r, pl.ds(c, _LANES)] = (
                            xbuf[k, b, r, pl.ds(c, _LANES)] + p
                        )
                    return carry3

                lax.fori_loop(0, D // _LANES, col, 0)

        for k in range(_NBUF):
            start_in(k, k)

        def group(g, carry):
            for k in range(_NBUF):
                ci = g * _NBUF + k
                wait_in(ci, k)

                @pl.when(g > 0)
                def _():
                    wait_out(ci - _NBUF, k)

                compute(k)
                start_out(ci, k)

                @pl.when(g < n_groups - 1)
                def _():
                    start_in(ci + _NBUF, k)

            return carry

        lax.fori_loop(0, n_groups, group, 0)
        done = n_groups * _NBUF
        rem = n_chunks - done
        last_out = [done - _NBUF + k for k in range(_NBUF)]
        for j in range(rem):
            ci = done + j
            k = ci % _NBUF
            wait_out(last_out[k], k)
            start_in(ci, k)
            wait_in(ci, k)
            compute(k)
            start_out(ci, k)
            last_out[k] = ci
        for k in range(_NBUF):
            wait_out(last_out[k], k)

    return sc_add


def kernel(x, pos_embedding):
    B, S, D = x.shape
    NW = 32
    rows_per_w = S // NW
    sc_add = _make_sc_add(B, S, D, rows_per_w)
    return sc_add(x, pos_embedding[:S])


# speedup vs baseline: 1.7955x; 1.0034x over previous
import functools

import jax
import jax.numpy as jnp
from jax import lax
from jax.experimental import pallas as pl
from jax.experimental.pallas import tpu as pltpu
from jax.experimental.pallas import tpu_sc as plsc

_LANES = 16
_NBUF = 3
_R = 4


def _make_sc_add(B, S, D, rows_per_w):
    R = _R
    n_chunks = rows_per_w // R
    n_groups = n_chunks // _NBUF
    mesh = plsc.VectorSubcoreMesh(core_axis_name="c", subcore_axis_name="s")
    NC = mesh.num_cores

    @functools.partial(
        pl.kernel,
        out_type=jax.ShapeDtypeStruct((B, S, D), jnp.float32),
        mesh=mesh,
        scratch_types=[
            pltpu.VMEM((_NBUF, R, D), jnp.float32),
            pltpu.VMEM((_NBUF, B, R, D), jnp.float32),
            pltpu.VMEM((_NBUF, B, R, D), jnp.float32),
        ]
        + [pltpu.SemaphoreType.DMA] * (2 * _NBUF),
    )
    def sc_add(x_hbm, pos_hbm, out_hbm, pbuf, xbuf, obuf, *sems):
        in_sems = sems[:_NBUF]
        out_sems = sems[_NBUF:]
        wid = lax.axis_index("s") * NC + lax.axis_index("c")
        base = wid * rows_per_w

        def in_descs(ci, k):
            s0 = base + ci * R
            return [
                pltpu.make_async_copy(
                    pos_hbm.at[pl.ds(s0, R)], pbuf.at[k], in_sems[k]
                ),
                pltpu.make_async_copy(
                    x_hbm.at[:, pl.ds(s0, R)], xbuf.at[k], in_sems[k]
                ),
            ]

        def out_descs(ci, k):
            s0 = base + ci * R
            return [
                pltpu.make_async_copy(
                    obuf.at[k], out_hbm.at[:, pl.ds(s0, R)], out_sems[k]
                )
            ]

        def start_in(ci, k):
            for d in in_descs(ci, k):
                d.start()

        def wait_in(ci, k):
            for d in in_descs(ci, k):
                d.wait()

        def start_out(ci, k):
            for d in out_descs(ci, k):
                d.start()

        def wait_out(ci, k):
            for d in out_descs(ci, k):
                d.wait()

        def compute(k):
            for r in range(R):

                def col(i, carry3):
                    c = i * _LANES
                    p = pbuf[k, r, pl.ds(c, _LANES)]
                    for b in range(B):
                        obuf[k, b, r, pl.ds(c, _LANES)] = (
                            xbuf[k, b, r, pl.ds(c, _LANES)] + p
                        )
                    return carry3

                lax.fori_loop(0, D // _LANES, col, 0)

        for k in range(_NBUF):
            start_in(k, k)

        def group(g, carry):
            for k in range(_NBUF):
                ci = g * _NBUF + k
                wait_in(ci, k)

                @pl.when(g > 0)
                def _():
                    wait_out(ci - _NBUF, k)

                compute(k)
                start_out(ci, k)

                @pl.when(g < n_groups - 1)
                def _():
                    start_in(ci + _NBUF, k)

            return carry

        lax.fori_loop(0, n_groups, group, 0)
        done = n_groups * _NBUF
        rem = n_chunks - done
        last_out = [done - _NBUF + k for k in range(_NBUF)]
        for j in range(rem):
            ci = done + j
            k = ci % _NBUF
            wait_out(last_out[k], k)
            start_in(ci, k)
            wait_in(ci, k)
            compute(k)
            start_out(ci, k)
            last_out[k] = ci
        for k in range(_NBUF):
            wait_out(last_out[k], k)

    return sc_add


def kernel(x, pos_embedding):
    B, S, D = x.shape
    NW = 32
    rows_per_w = S // NW
    sc_add = _make_sc_add(B, S, D, rows_per_w)
    return sc_add(x, pos_embedding[:S])
